# baseline (speedup 1.0000x reference)
import functools

import jax
import jax.numpy as jnp
from jax import lax
from jax.experimental import pallas as pl
from jax.experimental.pallas import tpu as pltpu
from jax.experimental.pallas import tpu_sc as plsc

B = 16384
D = 32
NC = 2
NS = 16
NW = NC * NS
BPW = B // NW
ICHUNK = 128
NCHUNK = BPW // ICHUNK
NBUF = 16

_MESH = plsc.VectorSubcoreMesh(core_axis_name="c", subcore_axis_name="s",
                               num_cores=NC, num_subcores=NS)


def _body(uid_hbm, mid_hbm, ut_hbm, mtf_hbm, wb_hbm, out_hbm,
          uidx_v, midx_v, mrows_v, tiles_v, wb_v, out_v,
          msem, *sems):
    wid = lax.axis_index("s") * NC + lax.axis_index("c")
    base = wid * BPW

    pltpu.sync_copy(uid_hbm.at[pl.ds(wid * NCHUNK, NCHUNK), :], uidx_v)
    pltpu.sync_copy(mid_hbm.at[pl.ds(wid * NCHUNK, NCHUNK), :], midx_v)
    pltpu.sync_copy(wb_hbm, wb_v)

    w0 = wb_v[pl.ds(0, 16)]
    w1 = wb_v[pl.ds(16, 16)]
    bias = wb_v[pl.ds(32, 16)]

    d_lo = lax.iota(jnp.int32, 16)

    def load_ids(ref, g):
        return ref[g // 8, pl.ds((g % 8) * 16, 16)]

    def fire(uid, slot):
        cb = pl.multiple_of((uid >> 7) * 128, 128)
        pltpu.async_copy(ut_hbm.at[:, pl.ds(cb, 128)],
                         tiles_v.at[slot], sems[slot])

    def extract(uid, i, slot):
        lane = jnp.full((16,), uid & 127, jnp.int32)
        slot_v = jnp.full((16,), slot, jnp.int32)
        cb = pl.multiple_of((uid >> 7) * 128, 128)
        pltpu.make_async_copy(ut_hbm.at[:, pl.ds(cb, 128)],
                              tiles_v.at[slot], sems[slot]).wait()
        u0 = plsc.load_gather(tiles_v, [slot_v, d_lo, lane])
        u1 = plsc.load_gather(tiles_v, [slot_v, d_lo + 16, lane])
        m0 = mrows_v[pl.ds(i * D, 16)]
        m1 = mrows_v[pl.ds(i * D + 16, 16)]
        t = (u0 * w0) * m0 + (u1 * w1) * m1
        mrows_v[pl.ds(i * D, 16)] = plsc.cumsum(t)

    def movie_group(g, _):
        mids = load_ids(midx_v, g)
        for k in range(16):
            i = g * 16 + k
            row = mids[k] * D
            pltpu.async_copy(mtf_hbm.at[pl.ds(row, D)],
                             mrows_v.at[pl.ds(i * D, D)], msem)

        @pl.when(g >= 4)
        def _():
            pltpu.make_async_copy(mtf_hbm.at[pl.ds(0, 16 * D)],
                                  mrows_v.at[pl.ds(0, 16 * D)], msem).wait()
        return 0

    lax.fori_loop(0, BPW // 16, movie_group, 0)
    pltpu.make_async_copy(mtf_hbm.at[pl.ds(0, 4 * 16 * D)],
                          mrows_v.at[pl.ds(0, 4 * 16 * D)], msem).wait()

    NGRP = BPW // 16
    ids0 = load_ids(uidx_v, 0)
    for k in range(NBUF):
        fire(ids0[k], k)

    def step(g, _):
        ids_g = load_ids(uidx_v, g)
        for k in range(16):
            extract(ids_g[k], g * 16 + k, k)

            @pl.when(g + 1 < NGRP)
            def _():
                ids_n = load_ids(uidx_v, jnp.minimum(g + 1, NGRP - 1))
                fire(ids_n[k], k)
        return 0

    lax.fori_loop(0, NGRP, step, 0)

    lanes = lax.iota(jnp.int32, 16)

    def collect(c, _):
        g = plsc.load_gather(mrows_v, [(c * 16 + lanes) * D + 15])
        out_v[pl.ds(c * 16, 16)] = g + bias
        return 0

    lax.fori_loop(0, BPW // 16, collect, 0, unroll=4)

    pltpu.sync_copy(out_v, out_hbm.at[pl.ds(base, BPW)])


@jax.jit
def _run(user_id, movie_id, user_table_t, movie_table_f, wb):
    f = pl.kernel(
        _body,
        out_type=jax.ShapeDtypeStruct((B,), jnp.float32),
        mesh=_MESH,
        compiler_params=pltpu.CompilerParams(needs_layout_passes=False,
                                             use_tc_tiling_on_sc=True),
        scratch_types=[
            pltpu.VMEM((NCHUNK, ICHUNK), jnp.int32),
            pltpu.VMEM((NCHUNK, ICHUNK), jnp.int32),
            pltpu.VMEM((BPW * D,), jnp.float32),
            pltpu.VMEM((NBUF, 32, 128), jnp.float32),
            pltpu.VMEM((48,), jnp.float32),
            pltpu.VMEM((BPW,), jnp.float32),
            pltpu.SemaphoreType.DMA,
        ] + [pltpu.SemaphoreType.DMA] * NBUF,
    )
    return f(user_id, movie_id, user_table_t, movie_table_f, wb)


def kernel(user_id, movie_id, user_table, movie_table, dense_w, dense_b):
    wb = jnp.concatenate(
        [dense_w.reshape(D), jnp.broadcast_to(dense_b, (16,))])
    out = _run(user_id.reshape(NW * NCHUNK, ICHUNK),
               movie_id.reshape(NW * NCHUNK, ICHUNK),
               user_table.T, movie_table.reshape(-1), wb)
    return out.reshape(B, 1)


# speedup vs baseline: 1.0130x; 1.0130x over previous
import functools

import jax
import jax.numpy as jnp
from jax import lax
from jax.experimental import pallas as pl
from jax.experimental.pallas import tpu as pltpu
from jax.experimental.pallas import tpu_sc as plsc

B = 16384
D = 32
NC = 2
NS = 16
NW = NC * NS
BPW = B // NW
ICHUNK = 128
NCHUNK = BPW // ICHUNK
NBUF = 16

_MESH = plsc.VectorSubcoreMesh(core_axis_name="c", subcore_axis_name="s",
                               num_cores=NC, num_subcores=NS)


def _body(uid_hbm, mid_hbm, ut_hbm, mtf_hbm, wb_hbm, out_hbm,
          uidx_v, midx_v, mrows_v, tiles_v, wb_v, out_v,
          msem, *sems):
    wid = lax.axis_index("s") * NC + lax.axis_index("c")
    base = wid * BPW

    pltpu.sync_copy(uid_hbm.at[pl.ds(wid * NCHUNK, NCHUNK), :], uidx_v)
    pltpu.sync_copy(mid_hbm.at[pl.ds(wid * NCHUNK, NCHUNK), :], midx_v)
    pltpu.sync_copy(wb_hbm, wb_v)

    w0 = wb_v[pl.ds(0, 16)]
    w1 = wb_v[pl.ds(16, 16)]
    bias = wb_v[pl.ds(32, 16)]

    d_lo = lax.iota(jnp.int32, 16)

    def load_ids(ref, g):
        return ref[g // 8, pl.ds((g % 8) * 16, 16)]

    def fire(uid, slot):
        cb = pl.multiple_of((uid >> 7) * 128, 128)
        pltpu.async_copy(ut_hbm.at[:, pl.ds(cb, 128)],
                         tiles_v.at[slot], sems[slot])

    def extract(uid, i, slot):
        lane = jnp.full((16,), uid & 127, jnp.int32)
        slot_v = jnp.full((16,), slot, jnp.int32)
        cb = pl.multiple_of((uid >> 7) * 128, 128)
        pltpu.make_async_copy(ut_hbm.at[:, pl.ds(cb, 128)],
                              tiles_v.at[slot], sems[slot]).wait()
        u0 = plsc.load_gather(tiles_v, [slot_v, d_lo, lane])
        u1 = plsc.load_gather(tiles_v, [slot_v, d_lo + 16, lane])
        m0 = mrows_v[pl.ds(i * D, 16)]
        m1 = mrows_v[pl.ds(i * D + 16, 16)]
        t = (u0 * w0) * m0 + (u1 * w1) * m1
        mrows_v[pl.ds(i * D, 16)] = plsc.cumsum(t)

    NGRP = BPW // 16
    ids0 = load_ids(uidx_v, 0)
    for k in range(NBUF):
        fire(ids0[k], k)

    def movie_group(g, _):
        mids = load_ids(midx_v, g)
        for k in range(16):
            i = g * 16 + k
            row = mids[k] * D
            pltpu.async_copy(mtf_hbm.at[pl.ds(row, D)],
                             mrows_v.at[pl.ds(i * D, D)], msem)

        @pl.when(g >= 4)
        def _():
            pltpu.make_async_copy(mtf_hbm.at[pl.ds(0, 16 * D)],
                                  mrows_v.at[pl.ds(0, 16 * D)], msem).wait()
        return 0

    lax.fori_loop(0, BPW // 16, movie_group, 0)
    pltpu.make_async_copy(mtf_hbm.at[pl.ds(0, 4 * 16 * D)],
                          mrows_v.at[pl.ds(0, 4 * 16 * D)], msem).wait()

    def step(g, _):
        ids_g = load_ids(uidx_v, g)
        for k in range(16):
            extract(ids_g[k], g * 16 + k, k)

            @pl.when(g + 1 < NGRP)
            def _():
                ids_n = load_ids(uidx_v, jnp.minimum(g + 1, NGRP - 1))
                fire(ids_n[k], k)
        return 0

    lax.fori_loop(0, NGRP, step, 0)

    lanes = lax.iota(jnp.int32, 16)

    def collect(c, _):
        g = plsc.load_gather(mrows_v, [(c * 16 + lanes) * D + 15])
        out_v[pl.ds(c * 16, 16)] = g + bias
        return 0

    lax.fori_loop(0, BPW // 16, collect, 0, unroll=4)

    pltpu.sync_copy(out_v, out_hbm.at[pl.ds(base, BPW)])


@jax.jit
def _run(user_id, movie_id, user_table_t, movie_table_f, wb):
    f = pl.kernel(
        _body,
        out_type=jax.ShapeDtypeStruct((B,), jnp.float32),
        mesh=_MESH,
        compiler_params=pltpu.CompilerParams(needs_layout_passes=False,
                                             use_tc_tiling_on_sc=True),
        scratch_types=[
            pltpu.VMEM((NCHUNK, ICHUNK), jnp.int32),
            pltpu.VMEM((NCHUNK, ICHUNK), jnp.int32),
            pltpu.VMEM((BPW * D,), jnp.float32),
            pltpu.VMEM((NBUF, 32, 128), jnp.float32),
            pltpu.VMEM((48,), jnp.float32),
            pltpu.VMEM((BPW,), jnp.float32),
            pltpu.SemaphoreType.DMA,
        ] + [pltpu.SemaphoreType.DMA] * NBUF,
    )
    return f(user_id, movie_id, user_table_t, movie_table_f, wb)


def kernel(user_id, movie_id, user_table, movie_table, dense_w, dense_b):
    wb = jnp.concatenate(
        [dense_w.reshape(D), jnp.broadcast_to(dense_b, (16,))])
    out = _run(user_id.reshape(NW * NCHUNK, ICHUNK),
               movie_id.reshape(NW * NCHUNK, ICHUNK),
               user_table.T, movie_table.reshape(-1), wb)
    return out.reshape(B, 1)
